# baseline (speedup 1.0000x reference)
import jax
import jax.numpy as jnp
from jax import lax
from jax.experimental import pallas as pl
from jax.experimental.pallas import tpu as pltpu
from jax.experimental.pallas import tpu_sc as plsc

N_NODES = 10000
IN_DIM = 128
OUT_DIM = 128
NUM_HEADS = 8
HEAD_DIM = 16
HALF = OUT_DIM // 2
HEADS_PER_CORE = 4
ACC_W = 80

NC, NS, NLANE = 2, 16, 16
N_PAD = 10240
ROWS_PER_TILE = N_PAD // NS
E = 320000
EDGES_PER_TILE = 20480
E_PAD = NS * EDGES_PER_TILE
CHUNK = 128
N_CHUNKS = EDGES_PER_TILE // CHUNK


def _qkv_body(x_ref, wq_ref, wk_ref, wv_ref, kv_ref, q_ref):
    x = x_ref[...]
    dn = (((1,), (1,)), ((), ()))
    q_r = lax.dot_general(x, wq_ref[...], dn, preferred_element_type=jnp.float32)
    k_r = lax.dot_general(x, wk_ref[...], dn, preferred_element_type=jnp.float32)
    v_r = lax.dot_general(x, wv_ref[...], dn, preferred_element_type=jnp.float32)
    kv_ref[0, :, :HALF] = k_r[:, :HALF].astype(jnp.bfloat16)
    kv_ref[0, :, HALF:] = v_r[:, :HALF].astype(jnp.bfloat16)
    kv_ref[1, :, :HALF] = k_r[:, HALF:].astype(jnp.bfloat16)
    kv_ref[1, :, HALF:] = v_r[:, HALF:].astype(jnp.bfloat16)
    q_ref[0] = q_r[:, :HALF].astype(jnp.bfloat16)
    q_ref[1] = q_r[:, HALF:].astype(jnp.bfloat16)


def _qkv(x_pad, WQ, WK, WV):
    blk = 256
    grid = (N_PAD // blk,)
    bs_x = pl.BlockSpec((blk, IN_DIM), lambda i: (i, 0))
    bs_w = pl.BlockSpec((OUT_DIM, IN_DIM), lambda i: (0, 0))
    bs_kv = pl.BlockSpec((NC, blk, OUT_DIM), lambda i: (0, i, 0))
    bs_o = pl.BlockSpec((NC, blk, HALF), lambda i: (0, i, 0))
    return pl.pallas_call(
        _qkv_body, grid=grid,
        in_specs=[bs_x, bs_w, bs_w, bs_w],
        out_specs=[bs_kv, bs_o],
        out_shape=[jax.ShapeDtypeStruct((NC, N_PAD, OUT_DIM), jnp.bfloat16),
                   jax.ShapeDtypeStruct((NC, N_PAD, HALF), jnp.bfloat16)],
    )(x_pad, WQ, WK, WV)


def _edge_body(kv_hbm, q_hbm, src2_hbm, dst2_hbm, zero80_hbm,
               acc_out,
               is_all, id_all, kv_buf, q_buf, msg_buf,
               acc_sh, sem_g):
    c = lax.axis_index("c")
    s = lax.axis_index("s")
    rbase = s * ROWS_PER_TILE
    kv_half = kv_hbm.at[c]
    q_half = q_hbm.at[c]

    pltpu.sync_copy(zero80_hbm, acc_sh.at[pl.ds(rbase, ROWS_PER_TILE)])
    pltpu.sync_copy(zero80_hbm.at[pl.ds(0, CHUNK)], msg_buf)
    pltpu.sync_copy(src2_hbm.at[pl.ds(s * N_CHUNKS, N_CHUNKS)], is_all)
    pltpu.sync_copy(dst2_hbm.at[pl.ds(s * N_CHUNKS, N_CHUNKS)], id_all)
    plsc.subcore_barrier()

    def fire(g):
        pltpu.async_copy(kv_half.at[is_all.at[g]], kv_buf, sem_g)
        pltpu.async_copy(q_half.at[id_all.at[g]], q_buf, sem_g)

    def wait_gather(g):
        pltpu.make_async_copy(kv_half.at[is_all.at[g]], kv_buf, sem_g).wait()
        pltpu.make_async_copy(q_half.at[id_all.at[g]], q_buf, sem_g).wait()

    lane = lax.iota(jnp.int32, NLANE)
    _perms = [lane ^ k for k in (1, 2, 4)]
    _zero_i = lane * 0
    _eight_i = _zero_i + 8

    def _halfsum(v):
        for p in _perms:
            v = v + v.at[p].get(mode="promise_in_bounds")
        return v

    def compute_chunk():
        @plsc.parallel_loop(0, CHUNK, unroll=8)
        def _edge_i(e):
            zvec = jnp.zeros((NLANE,), jnp.float32)
            for p in range(2):
                kk = kv_buf[e, pl.ds(p * 2 * HEAD_DIM, 2 * HEAD_DIM)]
                qq = q_buf[e, pl.ds(p * 2 * HEAD_DIM, 2 * HEAD_DIM)]
                ka, kb = plsc.unpack(kk, format=plsc.PackFormat.INTERLEAVED,
                                     preferred_element_type=jnp.float32)
                qa, qb = plsc.unpack(qq, format=plsc.PackFormat.INTERLEAVED,
                                     preferred_element_type=jnp.float32)
                r = _halfsum(ka * qa + kb * qb)
                sc01 = jnp.exp(jnp.clip(r * 0.25, -5.0, 5.0))
                s0 = sc01.at[_zero_i].get(mode="promise_in_bounds")
                s1 = sc01.at[_eight_i].get(mode="promise_in_bounds")
                vv = kv_buf[e, pl.ds(HALF + p * 2 * HEAD_DIM, 2 * HEAD_DIM)]
                va, vb = plsc.unpack(vv, format=plsc.PackFormat.INTERLEAVED,
                                     preferred_element_type=jnp.float32)
                msg_buf[e, pl.ds(p * 2 * HEAD_DIM, HEAD_DIM)] = va * sc01
                msg_buf[e, pl.ds(p * 2 * HEAD_DIM + HEAD_DIM, HEAD_DIM)] = vb * sc01
                zvec = jnp.where(lane == 2 * p, s0, zvec)
                zvec = jnp.where(lane == 2 * p + 1, s1, zvec)
            msg_buf[e, pl.ds(HALF, NLANE)] = zvec

    @pl.loop(0, N_CHUNKS)
    def _chunk(g):
        fire(g)
        wait_gather(g)
        compute_chunk()
        pltpu.sync_copy(msg_buf, acc_sh.at[id_all.at[g]], add=True)

    plsc.subcore_barrier()
    pltpu.sync_copy(acc_sh.at[pl.ds(rbase, ROWS_PER_TILE)],
                    acc_out.at[c, pl.ds(rbase, ROWS_PER_TILE)])


def _edge(kv, q, src2, dst2, zero80):
    mesh = plsc.VectorSubcoreMesh(core_axis_name="c", subcore_axis_name="s",
                                  num_cores=NC, num_subcores=NS)
    f32 = jnp.float32
    run = pl.kernel(
        _edge_body,
        out_type=jax.ShapeDtypeStruct((NC, N_PAD, ACC_W), f32),
        mesh=mesh,
        compiler_params=pltpu.CompilerParams(needs_layout_passes=False,
                                             use_tc_tiling_on_sc=False),
        scratch_types=[
            pltpu.VMEM((N_CHUNKS, CHUNK), jnp.int32),
            pltpu.VMEM((N_CHUNKS, CHUNK), jnp.int32),
            pltpu.VMEM((CHUNK, OUT_DIM), jnp.bfloat16),
            pltpu.VMEM((CHUNK, HALF), jnp.bfloat16),
            pltpu.VMEM((CHUNK, ACC_W), f32),
            pltpu.VMEM_SHARED((N_PAD, ACC_W), f32),
            pltpu.SemaphoreType.DMA,
        ],
    )
    return run(kv, q, src2, dst2, zero80)


def _norm_body(acc_ref, o_ref):
    a = acc_ref[...]
    wv = a[:, :HALF]
    zh = a[:, HALF:HALF + HEADS_PER_CORE]
    hr = lax.broadcasted_iota(jnp.int32, (HEADS_PER_CORE, HALF), 0)
    rc = lax.broadcasted_iota(jnp.int32, (HEADS_PER_CORE, HALF), 1)
    expand = (2 * (rc // 32) + (rc % 16) // 8 == hr).astype(jnp.float32)
    denom = lax.dot_general(zh, expand, (((1,), (0,)), ((), ())),
                            preferred_element_type=jnp.float32) + 1e-6
    rr_ = lax.broadcasted_iota(jnp.int32, (HALF, HALF), 0)
    cc_ = lax.broadcasted_iota(jnp.int32, (HALF, HALF), 1)
    r32 = rr_ % 32
    jj_ = r32 % 16
    orig = (rr_ // 32) * 32 + (jj_ // 8) * 16 + (jj_ % 8) * 2 + r32 // 16
    perm = (cc_ == orig).astype(jnp.float32)
    o_ref[...] = lax.dot_general(wv / denom, perm, (((1,), (0,)), ((), ())),
                                 preferred_element_type=jnp.float32)


def _norm(acc_flat):
    blk = 256
    grid = (NC * N_PAD // blk,)
    bs_a = pl.BlockSpec((blk, ACC_W), lambda i: (i, 0))
    bs_o = pl.BlockSpec((blk, HALF), lambda i: (i, 0))
    return pl.pallas_call(
        _norm_body, grid=grid,
        in_specs=[bs_a],
        out_specs=bs_o,
        out_shape=jax.ShapeDtypeStruct((NC * N_PAD, HALF), jnp.float32),
    )(acc_flat)


def kernel(x, edge_index, virt_h, virt_edge_index, WQ, WK, WV):
    x_pad = jnp.pad(x, ((0, N_PAD - N_NODES), (0, 0)))
    kv, q = _qkv(x_pad, WQ, WK, WV)

    src = edge_index[0].astype(jnp.int32)
    dst = edge_index[1].astype(jnp.int32)
    pad = jnp.full((E_PAD - E,), N_NODES, jnp.int32)
    src2 = jnp.concatenate([src, pad]).reshape(E_PAD // CHUNK, CHUNK)
    dst2 = jnp.concatenate([dst, pad]).reshape(E_PAD // CHUNK, CHUNK)

    zero80 = jnp.zeros((ROWS_PER_TILE, ACC_W), jnp.float32)
    acc = _edge(kv, q, src2, dst2, zero80)

    out_flat = _norm(acc.reshape(NC * N_PAD, ACC_W))
    return jnp.concatenate([out_flat[:N_NODES],
                            out_flat[N_PAD:N_PAD + N_NODES]], axis=1)


# speedup vs baseline: 1.1850x; 1.1850x over previous
import jax
import jax.numpy as jnp
from jax import lax
from jax.experimental import pallas as pl
from jax.experimental.pallas import tpu as pltpu
from jax.experimental.pallas import tpu_sc as plsc

N_NODES = 10000
IN_DIM = 128
OUT_DIM = 128
NUM_HEADS = 8
HEAD_DIM = 16
HALF = OUT_DIM // 2
HEADS_PER_CORE = 4
ACC_W = 80

NC, NS, NLANE = 2, 16, 16
N_PAD = 10240
ROWS_PER_TILE = N_PAD // NS
E = 320000
EDGES_PER_TILE = 20480
E_PAD = NS * EDGES_PER_TILE
CHUNK = 128
N_CHUNKS = EDGES_PER_TILE // CHUNK


def _qkv_body(x_ref, wq_ref, wk_ref, wv_ref, k_ref, q_ref, v_ref):
    x = x_ref[...]
    dn = (((1,), (1,)), ((), ()))
    q_r = lax.dot_general(x, wq_ref[...], dn, preferred_element_type=jnp.float32)
    k_r = lax.dot_general(x, wk_ref[...], dn, preferred_element_type=jnp.float32)
    v_r = lax.dot_general(x, wv_ref[...], dn, preferred_element_type=jnp.float32)
    k_ref[0] = k_r[:, :HALF].astype(jnp.bfloat16)
    k_ref[1] = k_r[:, HALF:].astype(jnp.bfloat16)
    q_ref[0] = q_r[:, :HALF].astype(jnp.bfloat16)
    q_ref[1] = q_r[:, HALF:].astype(jnp.bfloat16)
    v_ref[0] = v_r[:, :HALF].astype(jnp.bfloat16)
    v_ref[1] = v_r[:, HALF:].astype(jnp.bfloat16)


def _qkv(x_pad, WQ, WK, WV):
    blk = 256
    grid = (N_PAD // blk,)
    bs_x = pl.BlockSpec((blk, IN_DIM), lambda i: (i, 0))
    bs_w = pl.BlockSpec((OUT_DIM, IN_DIM), lambda i: (0, 0))
    bs_o = pl.BlockSpec((NC, blk, HALF), lambda i: (0, i, 0))
    return pl.pallas_call(
        _qkv_body, grid=grid,
        in_specs=[bs_x, bs_w, bs_w, bs_w],
        out_specs=[bs_o, bs_o, bs_o],
        out_shape=[jax.ShapeDtypeStruct((NC, N_PAD, HALF), jnp.bfloat16),
                   jax.ShapeDtypeStruct((NC, N_PAD, HALF), jnp.bfloat16),
                   jax.ShapeDtypeStruct((NC, N_PAD, HALF), jnp.bfloat16)],
    )(x_pad, WQ, WK, WV)


def _edge_body(k_hbm, q_hbm, v_hbm, src2_hbm, dst2_hbm, zero80_hbm,
               acc_out,
               is_all, id_all, k_buf, q_buf, v_buf, msg_buf,
               acc_sh, sem_g, sem_s):
    c = lax.axis_index("c")
    s = lax.axis_index("s")
    rbase = s * ROWS_PER_TILE
    k_half = k_hbm.at[c]
    q_half = q_hbm.at[c]
    v_half = v_hbm.at[c]

    pltpu.sync_copy(zero80_hbm, acc_sh.at[pl.ds(rbase, ROWS_PER_TILE)])
    pltpu.sync_copy(zero80_hbm.at[pl.ds(0, CHUNK)], msg_buf)
    pltpu.sync_copy(src2_hbm.at[pl.ds(s * N_CHUNKS, N_CHUNKS)], is_all)
    pltpu.sync_copy(dst2_hbm.at[pl.ds(s * N_CHUNKS, N_CHUNKS)], id_all)
    plsc.subcore_barrier()

    def fire(g):
        pltpu.async_copy(k_half.at[is_all.at[g]], k_buf, sem_g)
        pltpu.async_copy(q_half.at[id_all.at[g]], q_buf, sem_g)
        pltpu.async_copy(v_half.at[is_all.at[g]], v_buf, sem_g)

    def wait_gather(g):
        pltpu.make_async_copy(k_half.at[is_all.at[g]], k_buf, sem_g).wait()
        pltpu.make_async_copy(q_half.at[id_all.at[g]], q_buf, sem_g).wait()
        pltpu.make_async_copy(v_half.at[is_all.at[g]], v_buf, sem_g).wait()

    lane = lax.iota(jnp.int32, NLANE)
    _perms = [lane ^ k for k in (1, 2, 4)]
    _zero_i = lane * 0
    _eight_i = _zero_i + 8

    def _halfsum(v):
        for p in _perms:
            v = v + v.at[p].get(mode="promise_in_bounds")
        return v

    def compute_chunk():
        @plsc.parallel_loop(0, CHUNK, unroll=8)
        def _edge_i(e):
            zvec = jnp.zeros((NLANE,), jnp.float32)
            for p in range(2):
                kk = k_buf[e, pl.ds(p * 2 * HEAD_DIM, 2 * HEAD_DIM)]
                qq = q_buf[e, pl.ds(p * 2 * HEAD_DIM, 2 * HEAD_DIM)]
                ka, kb = plsc.unpack(kk, format=plsc.PackFormat.INTERLEAVED,
                                     preferred_element_type=jnp.float32)
                qa, qb = plsc.unpack(qq, format=plsc.PackFormat.INTERLEAVED,
                                     preferred_element_type=jnp.float32)
                r = _halfsum(ka * qa + kb * qb)
                sc01 = jnp.exp(jnp.clip(r * 0.25, -5.0, 5.0))
                s0 = sc01.at[_zero_i].get(mode="promise_in_bounds")
                s1 = sc01.at[_eight_i].get(mode="promise_in_bounds")
                vv = v_buf[e, pl.ds(p * 2 * HEAD_DIM, 2 * HEAD_DIM)]
                va, vb = plsc.unpack(vv, format=plsc.PackFormat.INTERLEAVED,
                                     preferred_element_type=jnp.float32)
                msg_buf[e, pl.ds(p * 2 * HEAD_DIM, HEAD_DIM)] = va * sc01
                msg_buf[e, pl.ds(p * 2 * HEAD_DIM + HEAD_DIM, HEAD_DIM)] = vb * sc01
                zvec = jnp.where(lane == 2 * p, s0, zvec)
                zvec = jnp.where(lane == 2 * p + 1, s1, zvec)
            msg_buf[e, pl.ds(HALF, NLANE)] = zvec

    pltpu.async_copy(msg_buf, acc_sh.at[id_all.at[jnp.int32(0)]], sem_s, add=True)

    @pl.loop(0, N_CHUNKS)
    def _chunk(g):
        fire(g)
        pltpu.make_async_copy(msg_buf, acc_sh.at[id_all.at[g]], sem_s).wait()
        wait_gather(g)
        compute_chunk()
        pltpu.async_copy(msg_buf, acc_sh.at[id_all.at[g]], sem_s, add=True)

    pltpu.make_async_copy(msg_buf, acc_sh.at[id_all.at[jnp.int32(0)]], sem_s).wait()
    plsc.subcore_barrier()
    pltpu.sync_copy(acc_sh.at[pl.ds(rbase, ROWS_PER_TILE)],
                    acc_out.at[c, pl.ds(rbase, ROWS_PER_TILE)])


def _edge(k, q, v, src2, dst2, zero80):
    mesh = plsc.VectorSubcoreMesh(core_axis_name="c", subcore_axis_name="s",
                                  num_cores=NC, num_subcores=NS)
    f32 = jnp.float32
    run = pl.kernel(
        _edge_body,
        out_type=jax.ShapeDtypeStruct((NC, N_PAD, ACC_W), f32),
        mesh=mesh,
        compiler_params=pltpu.CompilerParams(needs_layout_passes=False,
                                             use_tc_tiling_on_sc=False),
        scratch_types=[
            pltpu.VMEM((N_CHUNKS, CHUNK), jnp.int32),
            pltpu.VMEM((N_CHUNKS, CHUNK), jnp.int32),
            pltpu.VMEM((CHUNK, HALF), jnp.bfloat16),
            pltpu.VMEM((CHUNK, HALF), jnp.bfloat16),
            pltpu.VMEM((CHUNK, HALF), jnp.bfloat16),
            pltpu.VMEM((CHUNK, ACC_W), f32),
            pltpu.VMEM_SHARED((N_PAD, ACC_W), f32),
            pltpu.SemaphoreType.DMA,
            pltpu.SemaphoreType.DMA,
        ],
    )
    return run(k, q, v, src2, dst2, zero80)


def _norm_body(acc_ref, o_ref):
    a = acc_ref[...]
    wv = a[:, :HALF]
    zh = a[:, HALF:HALF + HEADS_PER_CORE]
    hr = lax.broadcasted_iota(jnp.int32, (HEADS_PER_CORE, HALF), 0)
    rc = lax.broadcasted_iota(jnp.int32, (HEADS_PER_CORE, HALF), 1)
    expand = (2 * (rc // 32) + (rc % 16) // 8 == hr).astype(jnp.float32)
    denom = lax.dot_general(zh, expand, (((1,), (0,)), ((), ())),
                            preferred_element_type=jnp.float32) + 1e-6
    rr_ = lax.broadcasted_iota(jnp.int32, (HALF, HALF), 0)
    cc_ = lax.broadcasted_iota(jnp.int32, (HALF, HALF), 1)
    r32 = rr_ % 32
    jj_ = r32 % 16
    orig = (rr_ // 32) * 32 + (jj_ // 8) * 16 + (jj_ % 8) * 2 + r32 // 16
    perm = (cc_ == orig).astype(jnp.float32)
    o_ref[...] = lax.dot_general(wv / denom, perm, (((1,), (0,)), ((), ())),
                                 preferred_element_type=jnp.float32)


def _norm(acc_flat):
    blk = 256
    grid = (NC * N_PAD // blk,)
    bs_a = pl.BlockSpec((blk, ACC_W), lambda i: (i, 0))
    bs_o = pl.BlockSpec((blk, HALF), lambda i: (i, 0))
    return pl.pallas_call(
        _norm_body, grid=grid,
        in_specs=[bs_a],
        out_specs=bs_o,
        out_shape=jax.ShapeDtypeStruct((NC * N_PAD, HALF), jnp.float32),
    )(acc_flat)


def kernel(x, edge_index, virt_h, virt_edge_index, WQ, WK, WV):
    x_pad = jnp.pad(x, ((0, N_PAD - N_NODES), (0, 0)))
    k, q, v = _qkv(x_pad, WQ, WK, WV)

    src = edge_index[0].astype(jnp.int32)
    dst = edge_index[1].astype(jnp.int32)
    pad = jnp.full((E_PAD - E,), N_NODES, jnp.int32)
    src2 = jnp.concatenate([src, pad]).reshape(E_PAD // CHUNK, CHUNK)
    dst2 = jnp.concatenate([dst, pad]).reshape(E_PAD // CHUNK, CHUNK)

    zero80 = jnp.zeros((ROWS_PER_TILE, ACC_W), jnp.float32)
    acc = _edge(k, q, v, src2, dst2, zero80)

    out_flat = _norm(acc.reshape(NC * N_PAD, ACC_W))
    return jnp.concatenate([out_flat[:N_NODES],
                            out_flat[N_PAD:N_PAD + N_NODES]], axis=1)


# speedup vs baseline: 1.5856x; 1.3381x over previous
import jax
import jax.numpy as jnp
from jax import lax
from jax.experimental import pallas as pl
from jax.experimental.pallas import tpu as pltpu
from jax.experimental.pallas import tpu_sc as plsc

N_NODES = 10000
IN_DIM = 128
OUT_DIM = 128
NUM_HEADS = 8
HEAD_DIM = 16
HALF = OUT_DIM // 2
HEADS_PER_CORE = 4
ACC_W = 80

NC, NS, NLANE = 2, 16, 16
N_PAD = 10240
ROWS_PER_TILE = N_PAD // NS
E = 320000
EDGES_PER_TILE = 20480
E_PAD = NS * EDGES_PER_TILE
CHUNK = 128
N_CHUNKS = EDGES_PER_TILE // CHUNK


def _qkv_body(x_ref, wq_ref, wk_ref, wv_ref, k_ref, q_ref, v_ref):
    x = x_ref[...]
    dn = (((1,), (1,)), ((), ()))
    q_r = lax.dot_general(x, wq_ref[...], dn, preferred_element_type=jnp.float32)
    k_r = lax.dot_general(x, wk_ref[...], dn, preferred_element_type=jnp.float32)
    v_r = lax.dot_general(x, wv_ref[...], dn, preferred_element_type=jnp.float32)
    k_ref[0] = k_r[:, :HALF].astype(jnp.bfloat16)
    k_ref[1] = k_r[:, HALF:].astype(jnp.bfloat16)
    q_ref[0] = q_r[:, :HALF].astype(jnp.bfloat16)
    q_ref[1] = q_r[:, HALF:].astype(jnp.bfloat16)
    v_ref[0] = v_r[:, :HALF].astype(jnp.bfloat16)
    v_ref[1] = v_r[:, HALF:].astype(jnp.bfloat16)


def _qkv(x_pad, WQ, WK, WV):
    blk = 256
    grid = (N_PAD // blk,)
    bs_x = pl.BlockSpec((blk, IN_DIM), lambda i: (i, 0))
    bs_w = pl.BlockSpec((OUT_DIM, IN_DIM), lambda i: (0, 0))
    bs_o = pl.BlockSpec((NC, blk, HALF), lambda i: (0, i, 0))
    return pl.pallas_call(
        _qkv_body, grid=grid,
        in_specs=[bs_x, bs_w, bs_w, bs_w],
        out_specs=[bs_o, bs_o, bs_o],
        out_shape=[jax.ShapeDtypeStruct((NC, N_PAD, HALF), jnp.bfloat16),
                   jax.ShapeDtypeStruct((NC, N_PAD, HALF), jnp.bfloat16),
                   jax.ShapeDtypeStruct((NC, N_PAD, HALF), jnp.bfloat16)],
    )(x_pad, WQ, WK, WV)


def _edge_body(k_hbm, q_hbm, v_hbm, src2_hbm, dst2_hbm, zero80_hbm,
               acc_out,
               is_all, id_all, k_a, k_b, q_a, q_b, v_a, v_b, msg_buf,
               acc_sh, sem_a, sem_b, sem_s):
    c = lax.axis_index("c")
    s = lax.axis_index("s")
    rbase = s * ROWS_PER_TILE
    k_half = k_hbm.at[c]
    q_half = q_hbm.at[c]
    v_half = v_hbm.at[c]

    pltpu.sync_copy(zero80_hbm, acc_sh.at[pl.ds(rbase, ROWS_PER_TILE)])
    pltpu.sync_copy(zero80_hbm.at[pl.ds(0, CHUNK)], msg_buf)
    pltpu.sync_copy(src2_hbm.at[pl.ds(s * N_CHUNKS, N_CHUNKS)], is_all)
    pltpu.sync_copy(dst2_hbm.at[pl.ds(s * N_CHUNKS, N_CHUNKS)], id_all)
    plsc.subcore_barrier()

    def fire(g, kd, qd, vd, sem):
        pltpu.async_copy(k_half.at[is_all.at[g]], kd, sem)
        pltpu.async_copy(q_half.at[id_all.at[g]], qd, sem)
        pltpu.async_copy(v_half.at[is_all.at[g]], vd, sem)

    def wait_gather(g, kd, qd, vd, sem):
        pltpu.make_async_copy(k_half.at[is_all.at[g]], kd, sem).wait()
        pltpu.make_async_copy(q_half.at[id_all.at[g]], qd, sem).wait()
        pltpu.make_async_copy(v_half.at[is_all.at[g]], vd, sem).wait()

    lane = lax.iota(jnp.int32, NLANE)
    _perms = [lane ^ k for k in (1, 2, 4)]
    _zero_i = lane * 0
    _eight_i = _zero_i + 8

    def _halfsum(v):
        for p in _perms:
            v = v + v.at[p].get(mode="promise_in_bounds")
        return v

    def compute_chunk(k_buf, q_buf, v_buf):
        @plsc.parallel_loop(0, CHUNK, unroll=8)
        def _edge_i(e):
            zvec = jnp.zeros((NLANE,), jnp.float32)
            for p in range(2):
                kk = k_buf[e, pl.ds(p * 2 * HEAD_DIM, 2 * HEAD_DIM)]
                qq = q_buf[e, pl.ds(p * 2 * HEAD_DIM, 2 * HEAD_DIM)]
                ka, kb = plsc.unpack(kk, format=plsc.PackFormat.INTERLEAVED,
                                     preferred_element_type=jnp.float32)
                qa, qb = plsc.unpack(qq, format=plsc.PackFormat.INTERLEAVED,
                                     preferred_element_type=jnp.float32)
                r = _halfsum(ka * qa + kb * qb)
                sc01 = jnp.exp(jnp.clip(r * 0.25, -5.0, 5.0))
                s0 = sc01.at[_zero_i].get(mode="promise_in_bounds")
                s1 = sc01.at[_eight_i].get(mode="promise_in_bounds")
                vv = v_buf[e, pl.ds(p * 2 * HEAD_DIM, 2 * HEAD_DIM)]
                va, vb = plsc.unpack(vv, format=plsc.PackFormat.INTERLEAVED,
                                     preferred_element_type=jnp.float32)
                msg_buf[e, pl.ds(p * 2 * HEAD_DIM, HEAD_DIM)] = va * sc01
                msg_buf[e, pl.ds(p * 2 * HEAD_DIM + HEAD_DIM, HEAD_DIM)] = vb * sc01
                zvec = jnp.where(lane == 2 * p, s0, zvec)
                zvec = jnp.where(lane == 2 * p + 1, s1, zvec)
            msg_buf[e, pl.ds(HALF, NLANE)] = zvec

    pltpu.async_copy(msg_buf, acc_sh.at[id_all.at[jnp.int32(0)]], sem_s, add=True)
    fire(jnp.int32(0), k_a, q_a, v_a, sem_a)

    @pl.loop(0, N_CHUNKS // 2)
    def _pair(gg):
        g0 = gg * 2
        g1 = gg * 2 + 1
        fire(g1, k_b, q_b, v_b, sem_b)
        pltpu.make_async_copy(msg_buf, acc_sh.at[id_all.at[g0]], sem_s).wait()
        wait_gather(g0, k_a, q_a, v_a, sem_a)
        compute_chunk(k_a, q_a, v_a)
        pltpu.async_copy(msg_buf, acc_sh.at[id_all.at[g0]], sem_s, add=True)
        fire(lax.rem(g0 + 2, N_CHUNKS), k_a, q_a, v_a, sem_a)
        pltpu.make_async_copy(msg_buf, acc_sh.at[id_all.at[g1]], sem_s).wait()
        wait_gather(g1, k_b, q_b, v_b, sem_b)
        compute_chunk(k_b, q_b, v_b)
        pltpu.async_copy(msg_buf, acc_sh.at[id_all.at[g1]], sem_s, add=True)

    pltpu.make_async_copy(msg_buf, acc_sh.at[id_all.at[jnp.int32(0)]], sem_s).wait()
    wait_gather(jnp.int32(0), k_a, q_a, v_a, sem_a)
    plsc.subcore_barrier()
    pltpu.sync_copy(acc_sh.at[pl.ds(rbase, ROWS_PER_TILE)],
                    acc_out.at[c, pl.ds(rbase, ROWS_PER_TILE)])


def _edge(k, q, v, src2, dst2, zero80):
    mesh = plsc.VectorSubcoreMesh(core_axis_name="c", subcore_axis_name="s",
                                  num_cores=NC, num_subcores=NS)
    f32 = jnp.float32
    run = pl.kernel(
        _edge_body,
        out_type=jax.ShapeDtypeStruct((NC, N_PAD, ACC_W), f32),
        mesh=mesh,
        compiler_params=pltpu.CompilerParams(needs_layout_passes=False,
                                             use_tc_tiling_on_sc=False),
        scratch_types=[
            pltpu.VMEM((N_CHUNKS, CHUNK), jnp.int32),
            pltpu.VMEM((N_CHUNKS, CHUNK), jnp.int32),
            pltpu.VMEM((CHUNK, HALF), jnp.bfloat16),
            pltpu.VMEM((CHUNK, HALF), jnp.bfloat16),
            pltpu.VMEM((CHUNK, HALF), jnp.bfloat16),
            pltpu.VMEM((CHUNK, HALF), jnp.bfloat16),
            pltpu.VMEM((CHUNK, HALF), jnp.bfloat16),
            pltpu.VMEM((CHUNK, HALF), jnp.bfloat16),
            pltpu.VMEM((CHUNK, ACC_W), f32),
            pltpu.VMEM_SHARED((N_PAD, ACC_W), f32),
            pltpu.SemaphoreType.DMA,
            pltpu.SemaphoreType.DMA,
            pltpu.SemaphoreType.DMA,
        ],
    )
    return run(k, q, v, src2, dst2, zero80)


def _norm_body(acc_ref, o_ref):
    a = acc_ref[...]
    wv = a[:, :HALF]
    zh = a[:, HALF:HALF + HEADS_PER_CORE]
    hr = lax.broadcasted_iota(jnp.int32, (HEADS_PER_CORE, HALF), 0)
    rc = lax.broadcasted_iota(jnp.int32, (HEADS_PER_CORE, HALF), 1)
    expand = (2 * (rc // 32) + (rc % 16) // 8 == hr).astype(jnp.float32)
    denom = lax.dot_general(zh, expand, (((1,), (0,)), ((), ())),
                            preferred_element_type=jnp.float32) + 1e-6
    rr_ = lax.broadcasted_iota(jnp.int32, (HALF, HALF), 0)
    cc_ = lax.broadcasted_iota(jnp.int32, (HALF, HALF), 1)
    r32 = rr_ % 32
    jj_ = r32 % 16
    orig = (rr_ // 32) * 32 + (jj_ // 8) * 16 + (jj_ % 8) * 2 + r32 // 16
    perm = (cc_ == orig).astype(jnp.float32)
    o_ref[...] = lax.dot_general(wv / denom, perm, (((1,), (0,)), ((), ())),
                                 preferred_element_type=jnp.float32)


def _norm(acc_flat):
    blk = 256
    grid = (NC * N_PAD // blk,)
    bs_a = pl.BlockSpec((blk, ACC_W), lambda i: (i, 0))
    bs_o = pl.BlockSpec((blk, HALF), lambda i: (i, 0))
    return pl.pallas_call(
        _norm_body, grid=grid,
        in_specs=[bs_a],
        out_specs=bs_o,
        out_shape=jax.ShapeDtypeStruct((NC * N_PAD, HALF), jnp.float32),
    )(acc_flat)


def kernel(x, edge_index, virt_h, virt_edge_index, WQ, WK, WV):
    x_pad = jnp.pad(x, ((0, N_PAD - N_NODES), (0, 0)))
    k, q, v = _qkv(x_pad, WQ, WK, WV)

    src = edge_index[0].astype(jnp.int32)
    dst = edge_index[1].astype(jnp.int32)
    pad = jnp.full((E_PAD - E,), N_NODES, jnp.int32)
    src2 = jnp.concatenate([src, pad]).reshape(E_PAD // CHUNK, CHUNK)
    dst2 = jnp.concatenate([dst, pad]).reshape(E_PAD // CHUNK, CHUNK)

    zero80 = jnp.zeros((ROWS_PER_TILE, ACC_W), jnp.float32)
    acc = _edge(k, q, v, src2, dst2, zero80)

    out_flat = _norm(acc.reshape(NC * N_PAD, ACC_W))
    return jnp.concatenate([out_flat[:N_NODES],
                            out_flat[N_PAD:N_PAD + N_NODES]], axis=1)
